# baseline (speedup 1.0000x reference)
import functools

import jax
import jax.numpy as jnp
from jax import lax
from jax.experimental import pallas as pl
from jax.experimental.pallas import tpu as pltpu
from jax.experimental.pallas import tpu_sc as plsc

BATCH = 16384
FACTOR = 32

NUM_CORES = 2
NUM_SUBCORES = 16
NUM_WORKERS = NUM_CORES * NUM_SUBCORES
BPW = BATCH // NUM_WORKERS


def _sc_gather(user, item, embed_user, embed_item):
    mesh = plsc.VectorSubcoreMesh(
        core_axis_name="c", subcore_axis_name="s",
        num_cores=NUM_CORES, num_subcores=NUM_SUBCORES)

    @functools.partial(
        pl.kernel,
        out_type=(
            jax.ShapeDtypeStruct((BATCH, FACTOR), jnp.float32),
            jax.ShapeDtypeStruct((BATCH, FACTOR), jnp.float32),
        ),
        mesh=mesh,
        scratch_types=[
            pltpu.VMEM((BPW,), jnp.int32),
            pltpu.VMEM((BPW,), jnp.int32),
            pltpu.SemaphoreType.DMA,
            pltpu.SemaphoreType.DMA,
        ],
    )
    def k(user_hbm, item_hbm, eu_hbm, ei_hbm, uout_hbm, vout_hbm,
          uidx_v, iidx_v, usem, vsem):
        wid = lax.axis_index("s") * NUM_CORES + lax.axis_index("c")
        base = wid * BPW
        pltpu.sync_copy(user_hbm.at[pl.ds(base, BPW)], uidx_v)
        pltpu.sync_copy(item_hbm.at[pl.ds(base, BPW)], iidx_v)

        def body(g, carry):
            uvec = uidx_v[pl.ds(g * 16, 16)]
            ivec = iidx_v[pl.ds(g * 16, 16)]
            for j in range(16):
                b = base + g * 16 + j
                pltpu.async_copy(eu_hbm.at[pl.ds(uvec[j], 1)],
                                 uout_hbm.at[pl.ds(b, 1)], usem)
                pltpu.async_copy(ei_hbm.at[pl.ds(ivec[j], 1)],
                                 vout_hbm.at[pl.ds(b, 1)], vsem)
            return carry

        lax.fori_loop(0, BPW // 16, body, 0)
        pltpu.make_async_copy(
            uout_hbm.at[pl.ds(0, BPW)],
            uout_hbm.at[pl.ds(base, BPW)], usem).wait()
        pltpu.make_async_copy(
            vout_hbm.at[pl.ds(0, BPW)],
            vout_hbm.at[pl.ds(base, BPW)], vsem).wait()

    return k(user, item, embed_user, embed_item)


def _tc_body(u_ref, v_ref, w_ref, b_ref, o_ref):
    prod = u_ref[...] * v_ref[...]
    logits = jax.lax.dot_general(
        prod, w_ref[...], (((1,), (0,)), ((), ())),
        preferred_element_type=jnp.float32) + b_ref[0]
    o_ref[...] = jax.nn.sigmoid(logits)


def _tc_epilogue(u_rows, v_rows, W, b):
    grid = 8
    blk = BATCH // grid
    out = pl.pallas_call(
        _tc_body,
        grid=(grid,),
        in_specs=[
            pl.BlockSpec((blk, FACTOR), lambda i: (i, 0)),
            pl.BlockSpec((blk, FACTOR), lambda i: (i, 0)),
            pl.BlockSpec((FACTOR, 1), lambda i: (0, 0)),
            pl.BlockSpec(memory_space=pltpu.SMEM),
        ],
        out_specs=pl.BlockSpec((blk, 1), lambda i: (i, 0)),
        out_shape=jax.ShapeDtypeStruct((BATCH, 1), jnp.float32),
    )(u_rows, v_rows, W, b)
    return out.reshape(-1)


@jax.jit
def kernel(user, item, embed_user, embed_item, W, b):
    u_rows, v_rows = _sc_gather(user, item, embed_user, embed_item)
    return _tc_epilogue(u_rows, v_rows, W, b)


# speedup vs baseline: 2.3771x; 2.3771x over previous
import functools

import jax
import jax.numpy as jnp
from jax import lax
from jax.experimental import pallas as pl
from jax.experimental.pallas import tpu as pltpu
from jax.experimental.pallas import tpu_sc as plsc

BATCH = 16384
FACTOR = 32

NUM_CORES = 2
NUM_SUBCORES = 16
NUM_WORKERS = NUM_CORES * NUM_SUBCORES
BPW = BATCH // NUM_WORKERS
WAVE = 256
NWAVE = BPW // WAVE


def _sc_gather(user, item, embed_user, embed_item):
    mesh = plsc.VectorSubcoreMesh(
        core_axis_name="c", subcore_axis_name="s",
        num_cores=NUM_CORES, num_subcores=NUM_SUBCORES)

    @functools.partial(
        pl.kernel,
        out_type=(
            jax.ShapeDtypeStruct((BATCH, FACTOR), jnp.float32),
            jax.ShapeDtypeStruct((BATCH, FACTOR), jnp.float32),
        ),
        mesh=mesh,
        scratch_types=[
            pltpu.VMEM((BPW,), jnp.int32),
            pltpu.VMEM((BPW,), jnp.int32),
            pltpu.VMEM((WAVE, FACTOR), jnp.float32),
            pltpu.VMEM((WAVE, FACTOR), jnp.float32),
            pltpu.SemaphoreType.DMA,
            pltpu.SemaphoreType.DMA,
        ],
    )
    def k(user_hbm, item_hbm, eu_hbm, ei_hbm, uout_hbm, vout_hbm,
          uidx_v, iidx_v, urows_v, vrows_v, usem, vsem):
        wid = lax.axis_index("s") * NUM_CORES + lax.axis_index("c")
        base = wid * BPW
        pltpu.sync_copy(user_hbm.at[pl.ds(base, BPW)], uidx_v)
        pltpu.sync_copy(item_hbm.at[pl.ds(base, BPW)], iidx_v)

        def wave(w, carry):
            def body(g, carry):
                uvec = uidx_v[pl.ds(w * WAVE + g * 16, 16)]
                ivec = iidx_v[pl.ds(w * WAVE + g * 16, 16)]
                for j in range(16):
                    r = g * 16 + j
                    pltpu.async_copy(eu_hbm.at[pl.ds(uvec[j], 1)],
                                     urows_v.at[pl.ds(r, 1)], usem)
                    pltpu.async_copy(ei_hbm.at[pl.ds(ivec[j], 1)],
                                     vrows_v.at[pl.ds(r, 1)], vsem)
                return carry

            lax.fori_loop(0, WAVE // 16, body, 0)
            pltpu.make_async_copy(
                uout_hbm.at[pl.ds(0, WAVE)], urows_v, usem).wait()
            pltpu.make_async_copy(
                vout_hbm.at[pl.ds(0, WAVE)], vrows_v, vsem).wait()
            ob = base + w * WAVE
            pltpu.sync_copy(urows_v, uout_hbm.at[pl.ds(ob, WAVE)])
            pltpu.sync_copy(vrows_v, vout_hbm.at[pl.ds(ob, WAVE)])
            return carry

        lax.fori_loop(0, NWAVE, wave, 0)

    return k(user, item, embed_user, embed_item)


def _tc_body(u_ref, v_ref, w_ref, b_ref, o_ref):
    prod = u_ref[...] * v_ref[...]
    logits = jax.lax.dot_general(
        prod, w_ref[...], (((1,), (0,)), ((), ())),
        preferred_element_type=jnp.float32) + b_ref[0]
    o_ref[...] = jax.nn.sigmoid(logits)


def _tc_epilogue(u_rows, v_rows, W, b):
    grid = 8
    blk = BATCH // grid
    out = pl.pallas_call(
        _tc_body,
        grid=(grid,),
        in_specs=[
            pl.BlockSpec((blk, FACTOR), lambda i: (i, 0)),
            pl.BlockSpec((blk, FACTOR), lambda i: (i, 0)),
            pl.BlockSpec((FACTOR, 1), lambda i: (0, 0)),
            pl.BlockSpec(memory_space=pltpu.SMEM),
        ],
        out_specs=pl.BlockSpec((blk, 1), lambda i: (i, 0)),
        out_shape=jax.ShapeDtypeStruct((BATCH, 1), jnp.float32),
    )(u_rows, v_rows, W, b)
    return out.reshape(-1)


@jax.jit
def kernel(user, item, embed_user, embed_item, W, b):
    u_rows, v_rows = _sc_gather(user, item, embed_user, embed_item)
    return _tc_epilogue(u_rows, v_rows, W, b)
